# speedup vs baseline: 2.9711x; 2.9711x over previous
import functools

import jax
import jax.numpy as jnp
from jax import lax
from jax.experimental import pallas as pl
from jax.experimental.pallas import tpu as pltpu
from jax.experimental.pallas import tpu_sc as plsc

D = 128
NC, NS = 2, 16
NW = NC * NS
CH = 128


def _make_gather(b_total: int):
  b_per_w = b_total // NW
  nchunk = b_per_w // CH
  mesh = plsc.VectorSubcoreMesh(core_axis_name="c", subcore_axis_name="s")

  @functools.partial(
      pl.kernel,
      mesh=mesh,
      out_type=jax.ShapeDtypeStruct((b_total, D), jnp.float32),
      scratch_types=[
          pltpu.VMEM((nchunk, CH), jnp.int32),
          pltpu.VMEM((CH, D), jnp.float32),
          pltpu.SemaphoreType.DMA,
      ],
  )
  def gather(x_hbm, table_hbm, out_hbm, idx_v, rows_v, sem):
    wid = lax.axis_index("s") * NC + lax.axis_index("c")
    base = wid * b_per_w
    pltpu.sync_copy(x_hbm.at[wid], idx_v)

    def body(c, carry):
      pltpu.async_copy(table_hbm.at[idx_v.at[c]], rows_v, sem).wait()
      pltpu.sync_copy(rows_v, out_hbm.at[pl.ds(base + c * CH, CH)])
      return carry

    lax.fori_loop(0, nchunk, body, 0)

  return gather


_gather = _make_gather(4096 * 50)


def kernel(x, table):
  b, h = x.shape
  flat = x.reshape(NW, (b * h) // (NW * CH), CH)
  out = _gather(flat, table)
  return out.reshape(b, h, D)


# speedup vs baseline: 3.2655x; 1.0991x over previous
import functools

import jax
import jax.numpy as jnp
from jax import lax
from jax.experimental import pallas as pl
from jax.experimental.pallas import tpu as pltpu
from jax.experimental.pallas import tpu_sc as plsc

D = 128
NC, NS = 2, 16
NW = NC * NS
CH = 80
CPG = 5
G = CH * CPG


def _make_gather(b_total: int):
  b_per_w = b_total // NW
  nchunk = b_per_w // CH
  ngroup = b_per_w // G
  mesh = plsc.VectorSubcoreMesh(core_axis_name="c", subcore_axis_name="s")

  @functools.partial(
      pl.kernel,
      mesh=mesh,
      out_type=jax.ShapeDtypeStruct((b_total, D), jnp.float32),
      scratch_types=[
          pltpu.VMEM((nchunk, CH), jnp.int32),
          pltpu.VMEM((G, D), jnp.float32),
          pltpu.VMEM((G, D), jnp.float32),
          pltpu.SemaphoreType.DMA,
          pltpu.SemaphoreType.DMA,
          pltpu.SemaphoreType.DMA,
          pltpu.SemaphoreType.DMA,
      ],
  )
  def gather(x_hbm, table_hbm, out_hbm, idx_v, rows0, rows1, g0, g1, w0, w1):
    wid = lax.axis_index("s") * NC + lax.axis_index("c")
    base = wid * b_per_w
    rows = (rows0, rows1)
    gsem = (g0, g1)
    wsem = (w0, w1)
    pltpu.sync_copy(x_hbm.at[wid], idx_v)

    def start_group(g, buf):
      for j in range(CPG):
        c = g * CPG + j
        pltpu.async_copy(
            table_hbm.at[idx_v.at[c]],
            rows[buf].at[pl.ds(j * CH, CH)],
            gsem[buf],
        )

    def wait_group(g, buf):
      for j in range(CPG):
        c = g * CPG + j
        pltpu.make_async_copy(
            table_hbm.at[idx_v.at[c]],
            rows[buf].at[pl.ds(j * CH, CH)],
            gsem[buf],
        ).wait()

    def start_write(g, buf):
      pltpu.async_copy(rows[buf], out_hbm.at[pl.ds(base + g * G, G)],
                       wsem[buf])

    def wait_write(g, buf):
      pltpu.make_async_copy(rows[buf], out_hbm.at[pl.ds(base + g * G, G)],
                            wsem[buf]).wait()

    start_group(0, 0)
    start_group(1, 1)

    def outer(o, carry):
      for buf in range(2):
        g = o * 2 + buf
        wait_group(g, buf)
        start_write(g, buf)
      for buf in range(2):
        g = o * 2 + buf
        wait_write(g, buf)
        start_group(g + 2, buf)
      return carry

    lax.fori_loop(0, ngroup // 2 - 1, outer, 0)

    for buf in range(2):
      g = ngroup - 2 + buf
      wait_group(g, buf)
      start_write(g, buf)
    for buf in range(2):
      g = ngroup - 2 + buf
      wait_write(g, buf)

  return gather


_gather = _make_gather(4096 * 50)


def kernel(x, table):
  b, h = x.shape
  flat = x.reshape(NW, (b * h) // (NW * CH), CH)
  out = _gather(flat, table)
  return out.reshape(b, h, D)


# speedup vs baseline: 5.6818x; 1.7400x over previous
import functools

import jax
import jax.numpy as jnp
from jax import lax
from jax.experimental import pallas as pl
from jax.experimental.pallas import tpu as pltpu
from jax.experimental.pallas import tpu_sc as plsc

D = 128
NC, NS = 2, 16
NW = NC * NS
BPG = 8


def _make_gather(batch: int, hist: int):
  b_per_w = batch // NW
  ngroup = b_per_w // BPG
  mesh = plsc.VectorSubcoreMesh(core_axis_name="c", subcore_axis_name="s")

  @functools.partial(
      pl.kernel,
      mesh=mesh,
      out_type=jax.ShapeDtypeStruct((batch, hist, D), jnp.float32),
      scratch_types=[
          pltpu.VMEM((b_per_w, hist), jnp.int32),
          pltpu.VMEM((BPG, hist, D), jnp.float32),
          pltpu.VMEM((BPG, hist, D), jnp.float32),
          pltpu.SemaphoreType.DMA,
          pltpu.SemaphoreType.DMA,
          pltpu.SemaphoreType.DMA,
          pltpu.SemaphoreType.DMA,
      ],
  )
  def gather(x_hbm, table_hbm, out_hbm, idx_v, rows0, rows1, g0, g1, w0, w1):
    wid = lax.axis_index("s") * NC + lax.axis_index("c")
    base = wid * b_per_w
    rows = (rows0, rows1)
    gsem = (g0, g1)
    wsem = (w0, w1)
    pltpu.sync_copy(x_hbm.at[pl.ds(base, b_per_w)], idx_v)

    def start_group(g, buf):
      for j in range(BPG):
        pltpu.async_copy(
            table_hbm.at[idx_v.at[g * BPG + j]],
            rows[buf].at[j],
            gsem[buf],
        )

    def wait_group(g, buf):
      for j in range(BPG):
        pltpu.make_async_copy(
            table_hbm.at[idx_v.at[g * BPG + j]],
            rows[buf].at[j],
            gsem[buf],
        ).wait()

    def start_write(g, buf):
      pltpu.async_copy(rows[buf], out_hbm.at[pl.ds(base + g * BPG, BPG)],
                       wsem[buf])

    def wait_write(g, buf):
      pltpu.make_async_copy(rows[buf], out_hbm.at[pl.ds(base + g * BPG, BPG)],
                            wsem[buf]).wait()

    start_group(0, 0)
    start_group(1, 1)

    def outer(o, carry):
      for buf in range(2):
        g = o * 2 + buf
        wait_group(g, buf)
        start_write(g, buf)
      for buf in range(2):
        g = o * 2 + buf
        wait_write(g, buf)
        start_group(g + 2, buf)
      return carry

    lax.fori_loop(0, ngroup // 2 - 1, outer, 0)

    for buf in range(2):
      g = ngroup - 2 + buf
      wait_group(g, buf)
      start_write(g, buf)
    for buf in range(2):
      g = ngroup - 2 + buf
      wait_write(g, buf)

  return gather


_gather = _make_gather(4096, 50)


def kernel(x, table):
  return _gather(x, table)
